# baseline (speedup 1.0000x reference)
import functools

import jax
import jax.numpy as jnp
from jax import lax
from jax.experimental import pallas as pl
from jax.experimental.pallas import tpu as pltpu
from jax.experimental.pallas import tpu_sc as plsc

_NUM_CORES = 2
_NUM_SUBCORES = 16
_NUM_WORKERS = _NUM_CORES * _NUM_SUBCORES
_LANES = 16
_NBUF = 3


@functools.partial(jax.jit, static_argnames=("hist",))
def _gather_t(idx_flat, tab_t, *, hist):
    batch = idx_flat.shape[0] // hist
    d_model, vocab = tab_t.shape
    d_per_w = d_model // _NUM_WORKERS
    mesh = plsc.VectorSubcoreMesh(
        core_axis_name="c", subcore_axis_name="s",
        num_cores=_NUM_CORES, num_subcores=_NUM_SUBCORES,
    )

    @functools.partial(
        pl.kernel,
        out_type=jax.ShapeDtypeStruct((hist, d_model, batch), jnp.float32),
        mesh=mesh,
        scratch_types=[
            pltpu.VMEM((vocab,), jnp.float32),
            [pltpu.VMEM((batch,), jnp.int32) for _ in range(_NBUF)],
            [pltpu.VMEM((batch,), jnp.float32) for _ in range(_NBUF)],
            [pltpu.SemaphoreType.DMA for _ in range(_NBUF)],
            [pltpu.SemaphoreType.DMA for _ in range(_NBUF)],
        ],
        compiler_params=pltpu.CompilerParams(use_tc_tiling_on_sc=True,
                                             needs_layout_passes=False),
    )
    def k(idx_hbm, tabt_hbm, out_hbm, row_v, idx_vs, obufs, isems, wsems):
        wid = lax.axis_index("s") * _NUM_CORES + lax.axis_index("c")

        n_steps = d_per_w * hist
        depth = 2

        def idx_load(s):
            return pltpu.make_async_copy(
                idx_hbm.at[pl.ds((s % hist) * batch, batch)],
                idx_vs[s % _NBUF], isems[s % _NBUF])

        for s in range(min(depth, n_steps)):
            idx_load(s).start()
        pending = [None] * _NBUF
        for f in range(d_per_w):
            d = wid * d_per_w + f
            pltpu.sync_copy(tabt_hbm.at[d], row_v)
            for h in range(hist):
                step = f * hist + h
                ib = step % _NBUF
                ob = step % _NBUF
                if step + depth < n_steps:
                    idx_load(step + depth).start()
                idx_load(step).wait()
                if pending[ob] is not None:
                    pending[ob].wait()

                @plsc.parallel_loop(0, batch, step=_LANES, unroll=8)
                def body(i):
                    iv = idx_vs[ib][pl.ds(i, _LANES)]
                    obufs[ob][pl.ds(i, _LANES)] = plsc.load_gather(
                        row_v, [iv])

                w = pltpu.make_async_copy(obufs[ob], out_hbm.at[h, d],
                                          wsems[ob])
                w.start()
                pending[ob] = w
        for ob in range(_NBUF):
            if pending[ob] is not None:
                pending[ob].wait()

    return k(idx_flat, tab_t)


def kernel(indices, table):
    batch, hist = indices.shape
    idx_flat = indices.astype(jnp.int32).T.reshape(hist * batch)
    tab_t = table.T
    out_t = _gather_t(idx_flat, tab_t, hist=hist)
    return out_t.transpose(2, 0, 1)


# speedup vs baseline: 1.0272x; 1.0272x over previous
import functools

import jax
import jax.numpy as jnp
from jax import lax
from jax.experimental import pallas as pl
from jax.experimental.pallas import tpu as pltpu
from jax.experimental.pallas import tpu_sc as plsc

_NUM_CORES = 2
_NUM_SUBCORES = 16
_NUM_WORKERS = _NUM_CORES * _NUM_SUBCORES
_LANES = 16
_NIB = 4
_NOB = 2


@functools.partial(jax.jit, static_argnames=("hist",))
def _gather_t(idx_flat, tab_t, *, hist):
    batch = idx_flat.shape[0] // hist
    d_model, vocab = tab_t.shape
    d_per_w = d_model // _NUM_WORKERS
    mesh = plsc.VectorSubcoreMesh(
        core_axis_name="c", subcore_axis_name="s",
        num_cores=_NUM_CORES, num_subcores=_NUM_SUBCORES,
    )

    @functools.partial(
        pl.kernel,
        out_type=jax.ShapeDtypeStruct((hist, d_model, batch), jnp.float32),
        mesh=mesh,
        scratch_types=[
            pltpu.VMEM((vocab,), jnp.float32),
            [pltpu.VMEM((batch,), jnp.int32) for _ in range(_NIB)],
            [pltpu.VMEM((batch,), jnp.float32) for _ in range(_NOB)],
            [pltpu.SemaphoreType.DMA for _ in range(_NIB)],
            [pltpu.SemaphoreType.DMA for _ in range(_NOB)],
        ],
        compiler_params=pltpu.CompilerParams(use_tc_tiling_on_sc=True,
                                             needs_layout_passes=False),
    )
    def k(idx_hbm, tabt_hbm, out_hbm, row_v, idx_vs, obufs, isems, wsems):
        wid = lax.axis_index("s") * _NUM_CORES + lax.axis_index("c")

        n_steps = d_per_w * hist
        depth = 3

        def idx_load(s):
            return pltpu.make_async_copy(
                idx_hbm.at[pl.ds((s % hist) * batch, batch)],
                idx_vs[s % _NIB], isems[s % _NIB])

        for s in range(min(depth, n_steps)):
            idx_load(s).start()
        pending = [None] * _NOB
        for f in range(d_per_w):
            d = wid * d_per_w + f
            pltpu.sync_copy(tabt_hbm.at[d], row_v)
            for h in range(hist):
                step = f * hist + h
                ib = step % _NIB
                ob = step % _NOB
                if step + depth < n_steps:
                    idx_load(step + depth).start()
                idx_load(step).wait()
                if pending[ob] is not None:
                    pending[ob].wait()

                @plsc.parallel_loop(0, batch, step=_LANES, unroll=8)
                def body(i):
                    iv = idx_vs[ib][pl.ds(i, _LANES)]
                    obufs[ob][pl.ds(i, _LANES)] = plsc.load_gather(
                        row_v, [iv])

                w = pltpu.make_async_copy(obufs[ob], out_hbm.at[h, d],
                                          wsems[ob])
                w.start()
                pending[ob] = w
        for ob in range(_NOB):
            if pending[ob] is not None:
                pending[ob].wait()

    return k(idx_flat, tab_t)


def kernel(indices, table):
    batch, hist = indices.shape
    idx_flat = indices.astype(jnp.int32).T.reshape(hist * batch)
    tab_t = table.T
    out_t = _gather_t(idx_flat, tab_t, hist=hist)
    return out_t.transpose(2, 0, 1)
